# Initial kernel scaffold; baseline (speedup 1.0000x reference)
import jax
import jax.numpy as jnp
from jax.experimental import pallas as pl


def kernel(batch_reprs, batch_labels):
    raise NotImplementedError("write your pallas kernel here")


# speedup vs baseline: 16.5957x; 16.5957x over previous
import functools

import jax
import jax.numpy as jnp
import numpy as np
from jax.experimental import pallas as pl
from jax.experimental.pallas import tpu as pltpu

BS_ = 4096
D_ = 128
K_ = 10
MARGIN_ = 0.1
RB_ = 256
NB_ = BS_ // RB_
TOPK_ = K_ + 1


def _w_of_pos(p):
    return 1.0 / float(np.log2(p + 1.0)) + 1.0


def _loss_kernel(xblk_ref, lab_col_ref, sq_col_ref, yall_ref, lab_row_ref,
                 sq_row_ref, out_ref):
    i = pl.program_id(0)
    x = xblk_ref[...]
    y = yall_ref[...]
    lab_c = lab_col_ref[...]
    lab_r = lab_row_ref[...]
    sq_c = sq_col_ref[...]
    sq_r = sq_row_ref[...]

    g = jax.lax.dot_general(
        x, y, (((1,), (1,)), ((), ())),
        preferred_element_type=jnp.float32)
    d2 = sq_c + sq_r - 2.0 * g
    d2 = jnp.maximum(d2, 0.0)
    dist = jnp.sqrt(d2 + 1e-12)
    match = (lab_c == lab_r)
    match_f = match.astype(jnp.float32)
    sim_hat = (1.0 - match_f) * MARGIN_ - dist

    pos_num = jnp.sum(match_f, axis=1, keepdims=True)
    ks = jnp.minimum(pos_num, float(TOPK_))

    iota = jax.lax.broadcasted_iota(jnp.int32, (RB_, BS_), 1)
    neg_inf = jnp.float32(-jnp.inf)

    work = sim_hat
    fp_term = jnp.zeros((RB_, 1), jnp.float32)
    fp_num = jnp.zeros((RB_, 1), jnp.float32)
    for p in range(1, TOPK_ + 1):
        m = jnp.max(work, axis=1, keepdims=True)
        eq = work == m
        idx = jnp.min(jnp.where(eq, iota, BS_), axis=1, keepdims=True)
        sel = iota == idx
        flag = jnp.max(jnp.where(sel, match_f, 0.0), axis=1, keepdims=True)
        work = jnp.where(sel, neg_inf, work)
        cond = (float(p) <= ks) & (flag < 0.5)
        fp_term += jnp.where(cond, m * _w_of_pos(p), 0.0)
        fp_num += cond.astype(jnp.float32)

    workm = jnp.where(match, sim_hat, neg_inf)
    fn_term = jnp.zeros((RB_, 1), jnp.float32)
    lo = ks - fp_num
    for mi in range(TOPK_):
        v = jnp.max(workm, axis=1, keepdims=True)
        eq = workm == v
        idx = jnp.min(jnp.where(eq, iota, BS_), axis=1, keepdims=True)
        sel = iota == idx
        workm = jnp.where(sel, neg_inf, workm)
        cnt_gt = jnp.sum((sim_hat > v).astype(jnp.float32), axis=1,
                         keepdims=True)
        cnt_tie = jnp.sum(((sim_hat == v) & (iota < idx)).astype(jnp.float32),
                          axis=1, keepdims=True)
        rank = 1.0 + cnt_gt + cnt_tie
        w = 1.0 / (jnp.log(rank + 1.0) * (1.0 / np.log(2.0))) + 1.0
        mif = float(mi)
        selm = (mif >= lo) & (mif < ks)
        fn_term += jnp.where(selm, v * w, 0.0)

    blk = jnp.sum(fp_term - fn_term).reshape(1, 1)

    @pl.when(i == 0)
    def _():
        out_ref[...] = blk

    @pl.when(i > 0)
    def _():
        out_ref[...] += blk


@jax.jit
def kernel(batch_reprs, batch_labels):
    sq = jnp.sum(batch_reprs * batch_reprs, axis=1)
    sq_col = sq.reshape(BS_, 1)
    sq_row = sq.reshape(1, BS_)
    lab_col = batch_labels.reshape(BS_, 1)
    lab_row = batch_labels.reshape(1, BS_)

    out = pl.pallas_call(
        _loss_kernel,
        grid=(NB_,),
        in_specs=[
            pl.BlockSpec((RB_, D_), lambda i: (i, 0)),
            pl.BlockSpec((RB_, 1), lambda i: (i, 0)),
            pl.BlockSpec((RB_, 1), lambda i: (i, 0)),
            pl.BlockSpec((BS_, D_), lambda i: (0, 0)),
            pl.BlockSpec((1, BS_), lambda i: (0, 0)),
            pl.BlockSpec((1, BS_), lambda i: (0, 0)),
        ],
        out_specs=pl.BlockSpec((1, 1), lambda i: (0, 0)),
        out_shape=jax.ShapeDtypeStruct((1, 1), jnp.float32),
    )(batch_reprs, lab_col, sq_col, batch_reprs, lab_row, sq_row)
    return out[0, 0]


# speedup vs baseline: 48.4317x; 2.9183x over previous
import functools

import jax
import jax.numpy as jnp
import numpy as np
from jax.experimental import pallas as pl
from jax.experimental.pallas import tpu as pltpu

BS_ = 4096
D_ = 128
K_ = 10
MARGIN_ = 0.1
RB_ = 256
NB_ = BS_ // RB_
TOPK_ = K_ + 1


def _w_of_pos(p):
    return 1.0 / float(np.log2(p + 1.0)) + 1.0


def _loss_kernel(xblk_ref, lab_col_ref, sq_col_ref, yall_ref, lab_row_ref,
                 sq_row_ref, out_ref):
    i = pl.program_id(0)
    x = xblk_ref[...]
    y = yall_ref[...]
    lab_c = lab_col_ref[...]
    lab_r = lab_row_ref[...]
    sq_c = sq_col_ref[...]
    sq_r = sq_row_ref[...]

    g = jax.lax.dot_general(
        x, y, (((1,), (1,)), ((), ())),
        preferred_element_type=jnp.float32)
    d2 = sq_c + sq_r - 2.0 * g
    d2 = jnp.maximum(d2, 0.0)
    dist = jnp.sqrt(d2 + 1e-12)
    match = (lab_c == lab_r)
    match_f = match.astype(jnp.float32)
    sim_hat = (1.0 - match_f) * MARGIN_ - dist

    pos_num = jnp.sum(match_f, axis=1, keepdims=True)
    ks = jnp.minimum(pos_num, float(TOPK_))

    neg_inf = jnp.float32(-jnp.inf)


    work = sim_hat
    pos_term = jnp.zeros((RB_, 1), jnp.float32)
    for p in range(1, TOPK_ + 1):
        m = jnp.max(work, axis=1, keepdims=True)
        work = jnp.where(work == m, neg_inf, work)
        pos_term += jnp.where(float(p) <= ks, m * _w_of_pos(p), 0.0)

    workm = jnp.where(match, sim_hat, neg_inf)
    neg_term = jnp.zeros((RB_, 1), jnp.float32)
    for q in range(TOPK_):
        v = jnp.max(workm, axis=1, keepdims=True)
        workm = jnp.where(workm == v, neg_inf, workm)
        cnt_gt = jnp.sum((sim_hat > v).astype(jnp.float32), axis=1,
                         keepdims=True)
        rank = 1.0 + cnt_gt
        w = 1.0 / jnp.log2(rank + 1.0) + 1.0
        neg_term += jnp.where(float(q) < ks, v * w, 0.0)

    blk = jnp.sum(pos_term - neg_term).reshape(1, 1)

    @pl.when(i == 0)
    def _():
        out_ref[...] = blk

    @pl.when(i > 0)
    def _():
        out_ref[...] += blk


@jax.jit
def kernel(batch_reprs, batch_labels):
    sq = jnp.sum(batch_reprs * batch_reprs, axis=1)
    sq_col = sq.reshape(BS_, 1)
    sq_row = sq.reshape(1, BS_)
    lab_col = batch_labels.reshape(BS_, 1)
    lab_row = batch_labels.reshape(1, BS_)

    out = pl.pallas_call(
        _loss_kernel,
        grid=(NB_,),
        in_specs=[
            pl.BlockSpec((RB_, D_), lambda i: (i, 0)),
            pl.BlockSpec((RB_, 1), lambda i: (i, 0)),
            pl.BlockSpec((RB_, 1), lambda i: (i, 0)),
            pl.BlockSpec((BS_, D_), lambda i: (0, 0)),
            pl.BlockSpec((1, BS_), lambda i: (0, 0)),
            pl.BlockSpec((1, BS_), lambda i: (0, 0)),
        ],
        out_specs=pl.BlockSpec((1, 1), lambda i: (0, 0)),
        out_shape=jax.ShapeDtypeStruct((1, 1), jnp.float32),
    )(batch_reprs, lab_col, sq_col, batch_reprs, lab_row, sq_row)
    return out[0, 0]


# speedup vs baseline: 49.5399x; 1.0229x over previous
import functools

import jax
import jax.numpy as jnp
import numpy as np
from jax.experimental import pallas as pl
from jax.experimental.pallas import tpu as pltpu

BS_ = 4096
D_ = 128
K_ = 10
MARGIN_ = 0.1
RB_ = 256
NB_ = BS_ // RB_
TOPK_ = K_ + 1
WIN_ = 512
LMAX_ = WIN_ - RB_ - 127


def _w_of_pos(p):
    return 1.0 / float(np.log2(p + 1.0)) + 1.0


def _sim_block(x, y, sq_c, sq_r):
    g = jax.lax.dot_general(
        x, y, (((1,), (1,)), ((), ())),
        preferred_element_type=jnp.float32)
    d2 = jnp.maximum(sq_c + sq_r - 2.0 * g, 0.0)
    return jnp.sqrt(d2 + 1e-12)


def _pos_term(sim_hat, ks):
    work = sim_hat
    neg_inf = jnp.float32(-jnp.inf)
    pos_term = jnp.zeros((RB_, 1), jnp.float32)
    for p in range(1, TOPK_ + 1):
        m = jnp.max(work, axis=1, keepdims=True)
        work = jnp.where(work == m, neg_inf, work)
        pos_term += jnp.where(float(p) <= ks, m * _w_of_pos(p), 0.0)
    return pos_term


def _neg_term(workm, sim_hat, ks):
    neg_inf = jnp.float32(-jnp.inf)
    neg_term = jnp.zeros((RB_, 1), jnp.float32)
    for q in range(TOPK_):
        v = jnp.max(workm, axis=1, keepdims=True)
        workm = jnp.where(workm == v, neg_inf, workm)
        cnt_gt = jnp.sum((sim_hat > v).astype(jnp.float32), axis=1,
                         keepdims=True)
        w = 1.0 / jnp.log2(2.0 + cnt_gt) + 1.0
        neg_term += jnp.where(float(q) < ks, v * w, 0.0)
    return neg_term


def _accum_out(i, blk, out_ref):
    @pl.when(i == 0)
    def _():
        out_ref[...] = blk

    @pl.when(i > 0)
    def _():
        out_ref[...] += blk


def _full_kernel(xblk_ref, lab_col_ref, sq_col_ref, yall_ref, lab_row_ref,
                 sq_row_ref, out_ref):
    i = pl.program_id(0)
    dist = _sim_block(xblk_ref[...], yall_ref[...], sq_col_ref[...],
                      sq_row_ref[...])
    match = lab_col_ref[...] == lab_row_ref[...]
    match_f = match.astype(jnp.float32)
    sim_hat = (1.0 - match_f) * MARGIN_ - dist

    pos_num = jnp.sum(match_f, axis=1, keepdims=True)
    ks = jnp.minimum(pos_num, float(TOPK_))

    workm = jnp.where(match, sim_hat, jnp.float32(-jnp.inf))
    blk = jnp.sum(_pos_term(sim_hat, ks)
                  - _neg_term(workm, sim_hat, ks)).reshape(1, 1)
    _accum_out(i, blk, out_ref)


def _band_kernel(xblk_ref, lab_col_ref, sq_col_ref, pos_ref, yall_ref,
                 lab_row_ref, sq_row_ref, ywin_ref, lab_win_ref, sq_win_ref,
                 out_ref):
    i = pl.program_id(0)
    x = xblk_ref[...]
    lab_c = lab_col_ref[...]
    sq_c = sq_col_ref[...]
    dist = _sim_block(x, yall_ref[...], sq_c, sq_row_ref[...])
    match_f = (lab_c == lab_row_ref[...]).astype(jnp.float32)
    sim_hat = (1.0 - match_f) * MARGIN_ - dist

    ks = jnp.minimum(pos_ref[...], float(TOPK_))

    distw = _sim_block(x, ywin_ref[0], sq_c, sq_win_ref[0])
    workm = jnp.where(lab_c == lab_win_ref[0], -distw, jnp.float32(-jnp.inf))

    blk = jnp.sum(_pos_term(sim_hat, ks)
                  - _neg_term(workm, sim_hat, ks)).reshape(1, 1)
    _accum_out(i, blk, out_ref)


def _run_full(x, lab_col, sq_col, lab_row, sq_row, pos_col):
    del pos_col
    return pl.pallas_call(
        _full_kernel,
        grid=(NB_,),
        in_specs=[
            pl.BlockSpec((RB_, D_), lambda i: (i, 0)),
            pl.BlockSpec((RB_, 1), lambda i: (i, 0)),
            pl.BlockSpec((RB_, 1), lambda i: (i, 0)),
            pl.BlockSpec((BS_, D_), lambda i: (0, 0)),
            pl.BlockSpec((1, BS_), lambda i: (0, 0)),
            pl.BlockSpec((1, BS_), lambda i: (0, 0)),
        ],
        out_specs=pl.BlockSpec((1, 1), lambda i: (0, 0)),
        out_shape=jax.ShapeDtypeStruct((1, 1), jnp.float32),
    )(x, lab_col, sq_col, x, lab_row, sq_row)


def _run_band(x, lab_col, sq_col, lab_row, sq_row, pos_col):
    starts = [min(max(i * RB_ - 128, 0), BS_ - WIN_) for i in range(NB_)]
    ywin = jnp.stack([jax.lax.slice(x, (s, 0), (s + WIN_, D_))
                      for s in starts])
    labw = jnp.stack([jax.lax.slice(lab_row, (0, s), (1, s + WIN_))
                      for s in starts])
    sqw = jnp.stack([jax.lax.slice(sq_row, (0, s), (1, s + WIN_))
                     for s in starts])
    return pl.pallas_call(
        _band_kernel,
        grid=(NB_,),
        in_specs=[
            pl.BlockSpec((RB_, D_), lambda i: (i, 0)),
            pl.BlockSpec((RB_, 1), lambda i: (i, 0)),
            pl.BlockSpec((RB_, 1), lambda i: (i, 0)),
            pl.BlockSpec((RB_, 1), lambda i: (i, 0)),
            pl.BlockSpec((BS_, D_), lambda i: (0, 0)),
            pl.BlockSpec((1, BS_), lambda i: (0, 0)),
            pl.BlockSpec((1, BS_), lambda i: (0, 0)),
            pl.BlockSpec((1, WIN_, D_), lambda i: (i, 0, 0)),
            pl.BlockSpec((1, 1, WIN_), lambda i: (i, 0, 0)),
            pl.BlockSpec((1, 1, WIN_), lambda i: (i, 0, 0)),
        ],
        out_specs=pl.BlockSpec((1, 1), lambda i: (0, 0)),
        out_shape=jax.ShapeDtypeStruct((1, 1), jnp.float32),
    )(x, lab_col, sq_col, pos_col, x, lab_row, sq_row, ywin, labw, sqw)


@jax.jit
def kernel(batch_reprs, batch_labels):
    order = jnp.argsort(batch_labels)
    labs = batch_labels[order]
    x = batch_reprs[order]

    sq = jnp.sum(x * x, axis=1)
    idx = jnp.arange(BS_, dtype=jnp.int32)
    diff = labs[1:] != labs[:-1]
    new_run = jnp.concatenate([jnp.array([True]), diff])
    is_end = jnp.concatenate([diff, jnp.array([True])])
    run_start = jax.lax.cummax(jnp.where(new_run, idx, 0))
    run_end = jax.lax.cummin(jnp.where(is_end, idx, BS_ - 1)[::-1])[::-1]
    pos_num = ((run_end - run_start) + 1).astype(jnp.float32)
    l_max = jnp.max(pos_num)

    args = (x, labs.reshape(BS_, 1), sq.reshape(BS_, 1),
            labs.reshape(1, BS_), sq.reshape(1, BS_),
            pos_num.reshape(BS_, 1))
    out = jax.lax.cond(l_max <= float(LMAX_), _run_band, _run_full, *args)
    return out[0, 0]
